# Initial kernel scaffold; baseline (speedup 1.0000x reference)
import jax
import jax.numpy as jnp
from jax.experimental import pallas as pl


def kernel(x, edge_index, additional_x, lf_embedding, Wr1, Ws1, b1, Wr2, Ws2, b2, Wr3, Ws3, b3):
    raise NotImplementedError("write your pallas kernel here")


# speedup vs baseline: 2.8922x; 2.8922x over previous
import functools

import jax
import jax.numpy as jnp
from jax import lax
from jax.experimental import pallas as pl
from jax.experimental.pallas import tpu as pltpu
from jax.experimental.pallas import tpu_sc as plsc

N = 50000
E = 800000

NC = 2
NS = 16
NW = NC * NS
FP = 16
CH = 128
GRP = 8
GPW = 25
CPW = GRP * GPW
E_PAD = NW * CPW * CH
RPS = 3128
ACC_ROWS = NS * RPS
TRASH = N

BN = 2000
GRID = N // BN

_F32 = jnp.float32
_HI = lax.Precision.HIGHEST


def _dot(a, b):
    return jnp.dot(a, b, preferred_element_type=_F32, precision=_HI)


@functools.lru_cache(maxsize=None)
def _make_sc_agg(P):
    mesh = plsc.VectorSubcoreMesh(core_axis_name="c", subcore_axis_name="s",
                                  num_cores=NC, num_subcores=NS)

    @functools.partial(
        pl.kernel,
        out_type=jax.ShapeDtypeStruct((NC, P, ACC_ROWS, FP), _F32),
        mesh=mesh,
        scratch_types=[
            pltpu.VMEM((GRP, CH), jnp.int32),
            pltpu.VMEM((GRP, CH), jnp.int32),
            pltpu.VMEM((GRP, CH, FP), _F32),
            pltpu.VMEM((512, FP), _F32),
            pltpu.VMEM_SHARED((ACC_ROWS, FP), _F32),
            pltpu.SemaphoreType.DMA,
            pltpu.SemaphoreType.DMA,
        ],
        compiler_params=pltpu.CompilerParams(use_tc_tiling_on_sc=False),
    )
    def sc_agg(src_hbm, dst_hbm, *rest):
        ys = rest[:P]
        out = rest[P]
        sbuf, dbuf, rows, zbuf, acc, gsem, ssem = rest[P + 1:]

        c = lax.axis_index("c")
        s = lax.axis_index("s")
        w = s * NC + c
        base = s * RPS

        def zrow(i, carry):
            zbuf[i, :] = jnp.zeros((FP,), _F32)
            return carry

        lax.fori_loop(0, 512, zrow, 0)

        for p in range(P):
            yp = ys[p]
            for z in range(RPS // 512):
                pltpu.sync_copy(zbuf, acc.at[pl.ds(base + z * 512, 512)])
            rem = RPS % 512
            if rem:
                pltpu.sync_copy(zbuf.at[pl.ds(0, rem)],
                                acc.at[pl.ds(base + (RPS // 512) * 512, rem)])
            plsc.subcore_barrier()

            def grp(g, carry):
                row0 = w * CPW + g * GRP
                pltpu.sync_copy(src_hbm.at[pl.ds(row0, GRP)], sbuf)
                pltpu.sync_copy(dst_hbm.at[pl.ds(row0, GRP)], dbuf)
                gathers = [
                    pltpu.async_copy(yp.at[sbuf.at[j]], rows.at[j], gsem)
                    for j in range(GRP)
                ]
                for cp in gathers:
                    cp.wait()
                scatters = [
                    pltpu.async_copy(rows.at[j], acc.at[dbuf.at[j]], ssem,
                                     add=True)
                    for j in range(GRP)
                ]
                for cp in scatters:
                    cp.wait()
                return carry

            lax.fori_loop(0, GPW, grp, 0)
            plsc.subcore_barrier()
            pltpu.sync_copy(acc.at[pl.ds(base, RPS)],
                            out.at[c, p, pl.ds(base, RPS)])
            plsc.subcore_barrier()

    return sc_agg


def _sc_call(P, src2d, dst2d, *ys):
    return _make_sc_agg(P)(src2d, dst2d, *ys)


def _tc1_body(x_ref, ax_ref, wr_ref, ws_ref, b_ref, *outs):
    xb = x_ref[...]
    ab = ax_ref[...]
    wr = wr_ref[...]
    ws = ws_ref[...]
    y = _dot(xb, wr[0:64, :]) + _dot(ab, wr[64:96, :])
    r = _dot(xb, ws[0:64, :]) + _dot(ab, ws[64:96, :]) + b_ref[...]
    for q in range(4):
        outs[q][...] = y[:, q * FP:(q + 1) * FP]
    outs[4][...] = r


def _tc2_body(acc_ref, r_ref, lf_ref, wr_ref, ws_ref, b_ref, *outs):
    a = acc_ref[...]
    r = r_ref[...]
    lf = lf_ref[...]
    wr = wr_ref[...]
    ws = ws_ref[...]
    y = _dot(lf, wr[64:96, :])
    r2 = _dot(lf, ws[64:96, :]) + b_ref[...]
    for p in range(4):
        pre = jnp.maximum(a[0, p] + a[1, p] + r[:, p * FP:(p + 1) * FP], 0.0)
        y = y + _dot(pre, wr[p * FP:(p + 1) * FP, :])
        r2 = r2 + _dot(pre, ws[p * FP:(p + 1) * FP, :])
    for q in range(6):
        outs[q][...] = y[:, q * FP:(q + 1) * FP]
    outs[6][...] = r2


def _tc3_body(acc_ref, r_ref, wr_ref, ws_ref, b_ref, ax_ref, *outs):
    a = acc_ref[...]
    r = r_ref[...]
    wr = wr_ref[...]
    ws = ws_ref[...]
    y = None
    r3 = b_ref[...] + ax_ref[...]
    for p in range(6):
        h = jnp.maximum(a[0, p] + a[1, p] + r[:, p * FP:(p + 1) * FP], 0.0)
        d = _dot(h, wr[p * FP:(p + 1) * FP, :])
        y = d if y is None else y + d
        r3 = r3 + _dot(h, ws[p * FP:(p + 1) * FP, :])
    for q in range(2):
        outs[q][...] = y[:, q * FP:(q + 1) * FP]
    outs[2][...] = r3


def _tc4_body(acc_ref, r_ref, out_ref):
    a = acc_ref[...]
    r = r_ref[...]
    out_ref[:, 0:FP] = a[0, 0] + a[1, 0] + r[:, 0:FP]
    out_ref[:, FP:2 * FP] = a[0, 1] + a[1, 1] + r[:, FP:2 * FP]


def _row_spec(cols):
    return pl.BlockSpec((BN, cols), lambda i: (i, 0))


def _full_spec(rows, cols):
    return pl.BlockSpec((rows, cols), lambda i: (0, 0))


def _acc_spec(P):
    return pl.BlockSpec((NC, P, BN, FP), lambda i: (0, 0, i, 0))


def _piece_shapes(P):
    return [jax.ShapeDtypeStruct((N, FP), _F32) for _ in range(P)]


def _tc1(x, ax, wr, ws, b):
    return pl.pallas_call(
        _tc1_body,
        grid=(GRID,),
        in_specs=[_row_spec(64), _row_spec(32), _full_spec(96, 64),
                  _full_spec(96, 64), _full_spec(1, 64)],
        out_specs=[_row_spec(FP)] * 4 + [_row_spec(64)],
        out_shape=_piece_shapes(4) + [jax.ShapeDtypeStruct((N, 64), _F32)],
    )(x, ax, wr, ws, b)


def _tc2(acc, r1, lf, wr, ws, b):
    return pl.pallas_call(
        _tc2_body,
        grid=(GRID,),
        in_specs=[_acc_spec(4), _row_spec(64), _row_spec(32),
                  _full_spec(96, 96), _full_spec(96, 96), _full_spec(1, 96)],
        out_specs=[_row_spec(FP)] * 6 + [_row_spec(96)],
        out_shape=_piece_shapes(6) + [jax.ShapeDtypeStruct((N, 96), _F32)],
    )(acc, r1, lf, wr, ws, b)


def _tc3(acc, r2, wr, ws, b, ax):
    return pl.pallas_call(
        _tc3_body,
        grid=(GRID,),
        in_specs=[_acc_spec(6), _row_spec(96), _full_spec(96, 32),
                  _full_spec(96, 32), _full_spec(1, 32), _row_spec(32)],
        out_specs=[_row_spec(FP)] * 2 + [_row_spec(32)],
        out_shape=_piece_shapes(2) + [jax.ShapeDtypeStruct((N, 32), _F32)],
    )(acc, r2, wr, ws, b, ax)


def _tc4(acc, r3):
    return pl.pallas_call(
        _tc4_body,
        grid=(GRID,),
        in_specs=[_acc_spec(2), _row_spec(32)],
        out_specs=_row_spec(32),
        out_shape=jax.ShapeDtypeStruct((N, 32), _F32),
    )(acc, r3)


def kernel(x, edge_index, additional_x, lf_embedding,
           Wr1, Ws1, b1, Wr2, Ws2, b2, Wr3, Ws3, b3):
    src = edge_index[0]
    dst = edge_index[1]
    pad = E_PAD - E
    src2d = jnp.concatenate(
        [src, jnp.zeros((pad,), jnp.int32)]).reshape(NW * CPW, CH)
    dst2d = jnp.concatenate(
        [dst, jnp.full((pad,), TRASH, jnp.int32)]).reshape(NW * CPW, CH)

    y1 = _tc1(x, additional_x, Wr1, Ws1, b1.reshape(1, 64))
    acc1 = _sc_call(4, src2d, dst2d, *y1[:4])
    y2 = _tc2(acc1, y1[4], lf_embedding, Wr2, Ws2, b2.reshape(1, 96))
    acc2 = _sc_call(6, src2d, dst2d, *y2[:6])
    y3 = _tc3(acc2, y2[6], Wr3, Ws3, b3.reshape(1, 32), additional_x)
    acc3 = _sc_call(2, src2d, dst2d, *y3[:2])
    return _tc4(acc3, y3[2])


# speedup vs baseline: 3.2808x; 1.1344x over previous
import functools

import jax
import jax.numpy as jnp
from jax import lax
from jax.experimental import pallas as pl
from jax.experimental.pallas import tpu as pltpu
from jax.experimental.pallas import tpu_sc as plsc

N = 50000
E = 800000

NC = 2
NS = 16
NW = NC * NS
FP = 16
CH = 128
GRP = 10
NB = 20
CPW = GRP * NB
H_CHUNKS = (120, 80)
H_BATCH = (12, 8)
CPH = H_CHUNKS[0]
E_PAD = NW * CPW * CH
RPS = 3128
ACC_ROWS = NS * RPS
TRASH = N
N_TRASH = ACC_ROWS - N

BN = 2000
GRID = N // BN

_F32 = jnp.float32
_HI = lax.Precision.HIGHEST


def _dot(a, b):
    return jnp.dot(a, b, preferred_element_type=_F32, precision=_HI)


@functools.lru_cache(maxsize=None)
def _make_sc_agg(P):
    mesh = plsc.VectorSubcoreMesh(core_axis_name="c", subcore_axis_name="s",
                                  num_cores=NC, num_subcores=NS)

    @functools.partial(
        pl.kernel,
        out_type=jax.ShapeDtypeStruct((NC, P, ACC_ROWS, FP), _F32),
        mesh=mesh,
        scratch_types=[
            pltpu.VMEM((CPH, CH), jnp.int32),
            pltpu.VMEM((CPH, CH), jnp.int32),
            pltpu.VMEM((GRP, CH, FP), _F32),
            pltpu.VMEM((GRP, CH, FP), _F32),
            pltpu.VMEM((128, FP), _F32),
            pltpu.VMEM_SHARED((ACC_ROWS, FP), _F32),
            pltpu.SemaphoreType.DMA,
            pltpu.SemaphoreType.DMA,
            pltpu.SemaphoreType.DMA,
            pltpu.SemaphoreType.DMA,
        ],
        compiler_params=pltpu.CompilerParams(use_tc_tiling_on_sc=False),
    )
    def sc_agg(src_hbm, dst_hbm, *rest):
        ys = rest[:P]
        out = rest[P]
        (sidx, didx, rows0, rows1, zbuf, acc,
         gsem0, gsem1, ssem0, ssem1) = rest[P + 1:]

        c = lax.axis_index("c")
        s = lax.axis_index("s")
        w = s * NC + c
        base = s * RPS

        def zrow(i, carry):
            zbuf[i, :] = jnp.zeros((FP,), _F32)
            return carry

        lax.fori_loop(0, 128, zrow, 0)

        for p in range(P):
            yp = ys[p]
            for z in range(RPS // 128):
                pltpu.sync_copy(zbuf, acc.at[pl.ds(base + z * 128, 128)])
            rem = RPS % 128
            if rem:
                pltpu.sync_copy(zbuf.at[pl.ds(0, rem)],
                                acc.at[pl.ds(base + (RPS // 128) * 128, rem)])
            plsc.subcore_barrier()

            def pair(k, carry):
                r0 = k * (2 * GRP)
                g0 = [
                    pltpu.async_copy(yp.at[sidx.at[r0 + j]], rows0.at[j],
                                     gsem0)
                    for j in range(GRP)
                ]
                g1 = [
                    pltpu.async_copy(yp.at[sidx.at[r0 + GRP + j]],
                                     rows1.at[j], gsem1)
                    for j in range(GRP)
                ]
                for cp in g0:
                    cp.wait()
                s0 = [
                    pltpu.async_copy(rows0.at[j], acc.at[didx.at[r0 + j]],
                                     ssem0, add=True)
                    for j in range(GRP)
                ]
                for cp in g1:
                    cp.wait()
                s1 = [
                    pltpu.async_copy(rows1.at[j],
                                     acc.at[didx.at[r0 + GRP + j]],
                                     ssem1, add=True)
                    for j in range(GRP)
                ]
                for cp in s0:
                    cp.wait()
                for cp in s1:
                    cp.wait()
                return carry

            off = 0
            for h in range(len(H_CHUNKS)):
                hc = H_CHUNKS[h]
                row0 = w * CPW + off
                pltpu.sync_copy(src_hbm.at[pl.ds(row0, hc)],
                                sidx.at[pl.ds(0, hc)])
                pltpu.sync_copy(dst_hbm.at[pl.ds(row0, hc)],
                                didx.at[pl.ds(0, hc)])
                lax.fori_loop(0, H_BATCH[h] // 2, pair, 0)
                off += hc
            plsc.subcore_barrier()
            pltpu.sync_copy(acc.at[pl.ds(base, RPS)],
                            out.at[c, p, pl.ds(base, RPS)])
            plsc.subcore_barrier()

    return sc_agg


def _sc_call(P, src2d, dst2d, *ys):
    return _make_sc_agg(P)(src2d, dst2d, *ys)


def _tc1_body(x_ref, ax_ref, wr_ref, ws_ref, b_ref, *outs):
    xb = x_ref[...]
    ab = ax_ref[...]
    wr = wr_ref[...]
    ws = ws_ref[...]
    y = _dot(xb, wr[0:64, :]) + _dot(ab, wr[64:96, :])
    r = _dot(xb, ws[0:64, :]) + _dot(ab, ws[64:96, :]) + b_ref[...]
    for q in range(4):
        outs[q][...] = y[:, q * FP:(q + 1) * FP]
    outs[4][...] = r


def _tc2_body(acc_ref, r_ref, lf_ref, wr_ref, ws_ref, b_ref, *outs):
    a = acc_ref[...]
    r = r_ref[...]
    lf = lf_ref[...]
    wr = wr_ref[...]
    ws = ws_ref[...]
    y = _dot(lf, wr[64:96, :])
    r2 = _dot(lf, ws[64:96, :]) + b_ref[...]
    for p in range(4):
        pre = jnp.maximum(a[0, p] + a[1, p] + r[:, p * FP:(p + 1) * FP], 0.0)
        y = y + _dot(pre, wr[p * FP:(p + 1) * FP, :])
        r2 = r2 + _dot(pre, ws[p * FP:(p + 1) * FP, :])
    for q in range(6):
        outs[q][...] = y[:, q * FP:(q + 1) * FP]
    outs[6][...] = r2


def _tc3_body(acc_ref, r_ref, wr_ref, ws_ref, b_ref, ax_ref, *outs):
    a = acc_ref[...]
    r = r_ref[...]
    wr = wr_ref[...]
    ws = ws_ref[...]
    y = None
    r3 = b_ref[...] + ax_ref[...]
    for p in range(6):
        h = jnp.maximum(a[0, p] + a[1, p] + r[:, p * FP:(p + 1) * FP], 0.0)
        d = _dot(h, wr[p * FP:(p + 1) * FP, :])
        y = d if y is None else y + d
        r3 = r3 + _dot(h, ws[p * FP:(p + 1) * FP, :])
    for q in range(2):
        outs[q][...] = y[:, q * FP:(q + 1) * FP]
    outs[2][...] = r3


def _tc4_body(acc_ref, r_ref, out_ref):
    a = acc_ref[...]
    r = r_ref[...]
    out_ref[:, 0:FP] = a[0, 0] + a[1, 0] + r[:, 0:FP]
    out_ref[:, FP:2 * FP] = a[0, 1] + a[1, 1] + r[:, FP:2 * FP]


def _row_spec(cols):
    return pl.BlockSpec((BN, cols), lambda i: (i, 0))


def _full_spec(rows, cols):
    return pl.BlockSpec((rows, cols), lambda i: (0, 0))


def _acc_spec(P):
    return pl.BlockSpec((NC, P, BN, FP), lambda i: (0, 0, i, 0))


def _piece_shapes(P):
    return [jax.ShapeDtypeStruct((N, FP), _F32) for _ in range(P)]


def _tc1(x, ax, wr, ws, b):
    return pl.pallas_call(
        _tc1_body,
        grid=(GRID,),
        in_specs=[_row_spec(64), _row_spec(32), _full_spec(96, 64),
                  _full_spec(96, 64), _full_spec(1, 64)],
        out_specs=[_row_spec(FP)] * 4 + [_row_spec(64)],
        out_shape=_piece_shapes(4) + [jax.ShapeDtypeStruct((N, 64), _F32)],
    )(x, ax, wr, ws, b)


def _tc2(acc, r1, lf, wr, ws, b):
    return pl.pallas_call(
        _tc2_body,
        grid=(GRID,),
        in_specs=[_acc_spec(4), _row_spec(64), _row_spec(32),
                  _full_spec(96, 96), _full_spec(96, 96), _full_spec(1, 96)],
        out_specs=[_row_spec(FP)] * 6 + [_row_spec(96)],
        out_shape=_piece_shapes(6) + [jax.ShapeDtypeStruct((N, 96), _F32)],
    )(acc, r1, lf, wr, ws, b)


def _tc3(acc, r2, wr, ws, b, ax):
    return pl.pallas_call(
        _tc3_body,
        grid=(GRID,),
        in_specs=[_acc_spec(6), _row_spec(96), _full_spec(96, 32),
                  _full_spec(96, 32), _full_spec(1, 32), _row_spec(32)],
        out_specs=[_row_spec(FP)] * 2 + [_row_spec(32)],
        out_shape=_piece_shapes(2) + [jax.ShapeDtypeStruct((N, 32), _F32)],
    )(acc, r2, wr, ws, b, ax)


def _tc4(acc, r3):
    return pl.pallas_call(
        _tc4_body,
        grid=(GRID,),
        in_specs=[_acc_spec(2), _row_spec(32)],
        out_specs=_row_spec(32),
        out_shape=jax.ShapeDtypeStruct((N, 32), _F32),
    )(acc, r3)


def kernel(x, edge_index, additional_x, lf_embedding,
           Wr1, Ws1, b1, Wr2, Ws2, b2, Wr3, Ws3, b3):
    src = edge_index[0]
    dst = edge_index[1]
    pad = E_PAD - E
    pad_dst = TRASH + (jnp.arange(pad, dtype=jnp.int32) % N_TRASH)
    src2d = jnp.concatenate(
        [src, jnp.zeros((pad,), jnp.int32)]).reshape(NW * CPW, CH)
    dst2d = jnp.concatenate([dst, pad_dst]).reshape(NW * CPW, CH)

    y1 = _tc1(x, additional_x, Wr1, Ws1, b1.reshape(1, 64))
    acc1 = _sc_call(4, src2d, dst2d, *y1[:4])
    y2 = _tc2(acc1, y1[4], lf_embedding, Wr2, Ws2, b2.reshape(1, 96))
    acc2 = _sc_call(6, src2d, dst2d, *y2[:6])
    y3 = _tc3(acc2, y2[6], Wr3, Ws3, b3.reshape(1, 32), additional_x)
    acc3 = _sc_call(2, src2d, dst2d, *y3[:2])
    return _tc4(acc3, y3[2])


# speedup vs baseline: 3.8578x; 1.1759x over previous
import functools

import jax
import jax.numpy as jnp
from jax import lax
from jax.experimental import pallas as pl
from jax.experimental.pallas import tpu as pltpu
from jax.experimental.pallas import tpu_sc as plsc

N = 50000
E = 800000

NC = 2
NS = 16
NW = NC * NS
FP = 16
CH = 128
GRP = 10
NB = 20
CPW = GRP * NB
H_CHUNKS = (120, 80)
H_BATCH = (12, 8)
CPH = H_CHUNKS[0]
E_PAD = NW * CPW * CH
RPS = 3250
ACC_ROWS = NS * RPS
TRASH = N
N_TRASH = ACC_ROWS - N

BN = 2000
GRID = N // BN

_F32 = jnp.float32
_HI = lax.Precision.HIGHEST


def _dot(a, b):
    return jnp.dot(a, b, preferred_element_type=_F32, precision=_HI)


@functools.lru_cache(maxsize=None)
def _make_sc_agg(P):
    mesh = plsc.VectorSubcoreMesh(core_axis_name="c", subcore_axis_name="s",
                                  num_cores=NC, num_subcores=NS)

    @functools.partial(
        pl.kernel,
        out_type=jax.ShapeDtypeStruct((NC, P, ACC_ROWS, FP), _F32),
        mesh=mesh,
        scratch_types=[
            pltpu.VMEM((CPH, CH), jnp.int32),
            pltpu.VMEM((CPH, CH), jnp.int32),
            pltpu.VMEM((GRP, CH, FP), _F32),
            pltpu.VMEM((GRP, CH, FP), _F32),
            pltpu.VMEM((128, FP), _F32),
            pltpu.VMEM_SHARED((ACC_ROWS, FP), _F32),
            pltpu.SemaphoreType.DMA,
            pltpu.SemaphoreType.DMA,
            pltpu.SemaphoreType.DMA,
            pltpu.SemaphoreType.DMA,
        ],
        compiler_params=pltpu.CompilerParams(use_tc_tiling_on_sc=False),
    )
    def sc_agg(src_hbm, dst_hbm, *rest):
        ys = rest[:P]
        out = rest[P]
        (sidx, didx, rows0, rows1, zbuf, acc,
         gsem0, gsem1, ssem0, ssem1) = rest[P + 1:]

        c = lax.axis_index("c")
        s = lax.axis_index("s")
        w = s * NC + c
        base = s * RPS

        def zrow(i, carry):
            zbuf[i, :] = jnp.zeros((FP,), _F32)
            return carry

        lax.fori_loop(0, 128, zrow, 0)

        for p in range(P):
            yp = ys[p]
            for z in range(RPS // 128):
                pltpu.sync_copy(zbuf, acc.at[pl.ds(base + z * 128, 128)])
            rem = RPS % 128
            if rem:
                pltpu.sync_copy(zbuf.at[pl.ds(0, rem)],
                                acc.at[pl.ds(base + (RPS // 128) * 128, rem)])
            plsc.subcore_barrier()

            def pair(k, carry):
                r0 = k * (2 * GRP)
                g0 = [
                    pltpu.async_copy(yp.at[sidx.at[r0 + j]], rows0.at[j],
                                     gsem0)
                    for j in range(GRP)
                ]
                g1 = [
                    pltpu.async_copy(yp.at[sidx.at[r0 + GRP + j]],
                                     rows1.at[j], gsem1)
                    for j in range(GRP)
                ]
                for cp in g0:
                    cp.wait()
                s0 = [
                    pltpu.async_copy(rows0.at[j], acc.at[didx.at[r0 + j]],
                                     ssem0, add=True)
                    for j in range(GRP)
                ]
                for cp in g1:
                    cp.wait()
                s1 = [
                    pltpu.async_copy(rows1.at[j],
                                     acc.at[didx.at[r0 + GRP + j]],
                                     ssem1, add=True)
                    for j in range(GRP)
                ]
                for cp in s0:
                    cp.wait()
                for cp in s1:
                    cp.wait()
                return carry

            off = 0
            for h in range(len(H_CHUNKS)):
                hc = H_CHUNKS[h]
                row0 = w * CPW + off
                pltpu.sync_copy(src_hbm.at[pl.ds(row0, hc)],
                                sidx.at[pl.ds(0, hc)])
                pltpu.sync_copy(dst_hbm.at[pl.ds(row0, hc)],
                                didx.at[pl.ds(0, hc)])
                lax.fori_loop(0, H_BATCH[h] // 2, pair, 0)
                off += hc
            plsc.subcore_barrier()
            pltpu.sync_copy(acc.at[pl.ds(base, RPS)],
                            out.at[c, p, pl.ds(base, RPS)])
            plsc.subcore_barrier()

    return sc_agg


def _sc_call(P, src2d, dst2d, *ys):
    return _make_sc_agg(P)(src2d, dst2d, *ys)


BNR = BN // 8


def _ilv(y):
    c = jnp.concatenate([y[BNR * i:BNR * (i + 1), :] for i in range(8)],
                        axis=1)
    return c.reshape(1, BNR, 128)


def _dilv(a):
    a2 = a.reshape(BNR, 128)
    return jnp.concatenate([a2[:, FP * i:FP * (i + 1)] for i in range(8)],
                           axis=0)


def _tc1_body(x_ref, ax_ref, wr_ref, ws_ref, b_ref, *outs):
    xb = x_ref[...]
    ab = ax_ref[...]
    wr = wr_ref[...]
    ws = ws_ref[...]
    y = _dot(xb, wr[0:64, :]) + _dot(ab, wr[64:96, :])
    r = _dot(xb, ws[0:64, :]) + _dot(ab, ws[64:96, :]) + b_ref[...]
    for q in range(4):
        outs[q][...] = _ilv(y[:, q * FP:(q + 1) * FP])
    outs[4][...] = r


def _tc2_body(acc_ref, r_ref, lf_ref, wr_ref, ws_ref, b_ref, *outs):
    a = acc_ref[...]
    r = r_ref[...]
    lf = lf_ref[...]
    wr = wr_ref[...]
    ws = ws_ref[...]
    y = _dot(lf, wr[64:96, :])
    r2 = _dot(lf, ws[64:96, :]) + b_ref[...]
    for p in range(4):
        agg = _dilv(a[0, p] + a[1, p])
        pre = jnp.maximum(agg + r[:, p * FP:(p + 1) * FP], 0.0)
        y = y + _dot(pre, wr[p * FP:(p + 1) * FP, :])
        r2 = r2 + _dot(pre, ws[p * FP:(p + 1) * FP, :])
    for q in range(6):
        outs[q][...] = _ilv(y[:, q * FP:(q + 1) * FP])
    outs[6][...] = r2


def _tc3_body(acc_ref, r_ref, wr_ref, ws_ref, b_ref, ax_ref, *outs):
    a = acc_ref[...]
    r = r_ref[...]
    wr = wr_ref[...]
    ws = ws_ref[...]
    y = None
    r3 = b_ref[...] + ax_ref[...]
    for p in range(6):
        agg = _dilv(a[0, p] + a[1, p])
        h = jnp.maximum(agg + r[:, p * FP:(p + 1) * FP], 0.0)
        d = _dot(h, wr[p * FP:(p + 1) * FP, :])
        y = d if y is None else y + d
        r3 = r3 + _dot(h, ws[p * FP:(p + 1) * FP, :])
    for q in range(2):
        outs[q][...] = _ilv(y[:, q * FP:(q + 1) * FP])
    outs[2][...] = r3


def _tc4_body(acc_ref, r_ref, out_ref):
    a = acc_ref[...]
    r = r_ref[...]
    out_ref[:, 0:FP] = _dilv(a[0, 0] + a[1, 0]) + r[:, 0:FP]
    out_ref[:, FP:2 * FP] = _dilv(a[0, 1] + a[1, 1]) + r[:, FP:2 * FP]


def _row_spec(cols):
    return pl.BlockSpec((BN, cols), lambda i: (i, 0))


def _full_spec(rows, cols):
    return pl.BlockSpec((rows, cols), lambda i: (0, 0))


def _acc_spec(P):
    return pl.BlockSpec((NC, P, 1, BNR, 128), lambda i: (0, 0, i, 0, 0))


def _piece_spec():
    return pl.BlockSpec((1, BNR, 128), lambda i: (i, 0, 0))


def _piece_shapes(P):
    return [jax.ShapeDtypeStruct((GRID, BNR, 128), _F32) for _ in range(P)]


def _tc1(x, ax, wr, ws, b):
    return pl.pallas_call(
        _tc1_body,
        grid=(GRID,),
        in_specs=[_row_spec(64), _row_spec(32), _full_spec(96, 64),
                  _full_spec(96, 64), _full_spec(1, 64)],
        out_specs=[_piece_spec()] * 4 + [_row_spec(64)],
        out_shape=_piece_shapes(4) + [jax.ShapeDtypeStruct((N, 64), _F32)],
    )(x, ax, wr, ws, b)


def _tc2(acc, r1, lf, wr, ws, b):
    return pl.pallas_call(
        _tc2_body,
        grid=(GRID,),
        in_specs=[_acc_spec(4), _row_spec(64), _row_spec(32),
                  _full_spec(96, 96), _full_spec(96, 96), _full_spec(1, 96)],
        out_specs=[_piece_spec()] * 6 + [_row_spec(96)],
        out_shape=_piece_shapes(6) + [jax.ShapeDtypeStruct((N, 96), _F32)],
    )(acc, r1, lf, wr, ws, b)


def _tc3(acc, r2, wr, ws, b, ax):
    return pl.pallas_call(
        _tc3_body,
        grid=(GRID,),
        in_specs=[_acc_spec(6), _row_spec(96), _full_spec(96, 32),
                  _full_spec(96, 32), _full_spec(1, 32), _row_spec(32)],
        out_specs=[_piece_spec()] * 2 + [_row_spec(32)],
        out_shape=_piece_shapes(2) + [jax.ShapeDtypeStruct((N, 32), _F32)],
    )(acc, r2, wr, ws, b, ax)


def _tc4(acc, r3):
    return pl.pallas_call(
        _tc4_body,
        grid=(GRID,),
        in_specs=[_acc_spec(2), _row_spec(32)],
        out_specs=_row_spec(32),
        out_shape=jax.ShapeDtypeStruct((N, 32), _F32),
    )(acc, r3)


def kernel(x, edge_index, additional_x, lf_embedding,
           Wr1, Ws1, b1, Wr2, Ws2, b2, Wr3, Ws3, b3):
    src = edge_index[0]
    dst = edge_index[1]
    pad = E_PAD - E

    def sigma(v):
        return (v // 2000) * 2000 + 8 * (v % 250) + (v % 2000) // 250

    pad_dst = TRASH + (jnp.arange(pad, dtype=jnp.int32) % N_TRASH)
    src2d = jnp.concatenate(
        [sigma(src), jnp.zeros((pad,), jnp.int32)]).reshape(NW * CPW, CH)
    dst2d = jnp.concatenate(
        [sigma(dst), pad_dst]).reshape(NW * CPW, CH)

    def to_sc(yp):
        return yp.reshape(N, FP)

    def to_tc(acc, P):
        return acc.reshape(NC, P, ACC_ROWS // BN, BNR, 128)

    y1 = _tc1(x, additional_x, Wr1, Ws1, b1.reshape(1, 64))
    acc1 = _sc_call(4, src2d, dst2d, *[to_sc(yp) for yp in y1[:4]])
    y2 = _tc2(to_tc(acc1, 4), y1[4], lf_embedding, Wr2, Ws2,
              b2.reshape(1, 96))
    acc2 = _sc_call(6, src2d, dst2d, *[to_sc(yp) for yp in y2[:6]])
    y3 = _tc3(to_tc(acc2, 6), y2[6], Wr3, Ws3, b3.reshape(1, 32),
              additional_x)
    acc3 = _sc_call(2, src2d, dst2d, *[to_sc(yp) for yp in y3[:2]])
    return _tc4(to_tc(acc3, 2), y3[2])
